# baseline (speedup 1.0000x reference)
import functools

import jax
import jax.numpy as jnp
from jax import lax
from jax.experimental import pallas as pl
from jax.experimental.pallas import tpu as pltpu
from jax.experimental.pallas import tpu_sc as plsc

N = 32768
D = 128
E = 64
BT = 2048

NC = 2
NS = 16
L = 16
NW = NC * NS
PER_W = N // NW
GROUPS = PER_W // L


def _gelu_exact(x):
    return 0.5 * x * (1.0 + lax.erf(x * (2.0 ** -0.5)))


def _logits_block(x_ref, w1_ref, b1_ref, w2_ref, b2_ref, w3_ref, b3_ref,
                  eb_ref, temp_ref, out_ref):
    x = x_ref[...]
    h = jnp.dot(x, w1_ref[...]) + b1_ref[...]
    h = _gelu_exact(h)
    h = jnp.dot(h, w2_ref[...]) + b2_ref[...]
    h = _gelu_exact(h)
    lt = lax.dot_general(w3_ref[...], h, (((0,), (1,)), ((), ())))
    lt = lt + (b3_ref[...] + eb_ref[...])[:, None]
    inv_t = 1.0 / jnp.maximum(temp_ref[0], 0.1)
    out_ref[...] = lt * inv_t


def _tc_logits(routing_features, W1, b1, W2, b2, W3, b3, expert_bias,
               temperature):
    full = lambda i: (0, 0)
    return pl.pallas_call(
        _logits_block,
        grid=(N // BT,),
        in_specs=[
            pl.BlockSpec((BT, D), lambda i: (i, 0)),
            pl.BlockSpec((D, 2 * D), full),
            pl.BlockSpec((2 * D,), lambda i: (0,)),
            pl.BlockSpec((2 * D, D), full),
            pl.BlockSpec((D,), lambda i: (0,)),
            pl.BlockSpec((D, E), full),
            pl.BlockSpec((E,), lambda i: (0,)),
            pl.BlockSpec((E,), lambda i: (0,)),
            pl.BlockSpec((1,), lambda i: (0,)),
        ],
        out_specs=pl.BlockSpec((E, BT), lambda i: (0, i)),
        out_shape=jax.ShapeDtypeStruct((E, N), jnp.float32),
        compiler_params=pltpu.CompilerParams(
            dimension_semantics=("parallel",)),
    )(routing_features, W1, b1, W2, b2, W3, b3, expert_bias, temperature)


def _sc_top2_body(lt_hbm, i1_hbm, i2_hbm, w1_hbm, w2_hbm,
                  lv, i1v, i2v, w1v, w2v):
    wid = lax.axis_index("s") * NC + lax.axis_index("c")
    base = wid * PER_W
    pltpu.sync_copy(lt_hbm.at[:, pl.ds(base, PER_W)], lv)

    def group(g, carry):
        sl = pl.ds(pl.multiple_of(g * L, L), L)
        m1 = lv[0, sl]
        i1 = jnp.zeros((L,), jnp.int32)
        m2 = jnp.full((L,), -jnp.inf, jnp.float32)
        i2 = jnp.zeros((L,), jnp.int32)
        for e in range(1, E):
            v = lv[e, sl]
            ei = jnp.full((L,), e, jnp.int32)
            gt1 = v > m1
            gt2 = v > m2
            m2 = jnp.where(gt1, m1, jnp.where(gt2, v, m2))
            i2 = jnp.where(gt1, i1, jnp.where(gt2, ei, i2))
            m1 = jnp.where(gt1, v, m1)
            i1 = jnp.where(gt1, ei, i1)
        s = jnp.exp(m2 - m1)
        d = 1.0 + s
        i1v[sl] = i1
        i2v[sl] = i2
        w1v[sl] = 1.0 / d
        w2v[sl] = s / d
        return carry

    lax.fori_loop(0, GROUPS, group, 0)

    pltpu.sync_copy(i1v, i1_hbm.at[pl.ds(base, PER_W)])
    pltpu.sync_copy(i2v, i2_hbm.at[pl.ds(base, PER_W)])
    pltpu.sync_copy(w1v, w1_hbm.at[pl.ds(base, PER_W)])
    pltpu.sync_copy(w2v, w2_hbm.at[pl.ds(base, PER_W)])


@functools.cache
def _sc_top2():
    return pl.kernel(
        _sc_top2_body,
        out_type=[
            jax.ShapeDtypeStruct((N,), jnp.int32),
            jax.ShapeDtypeStruct((N,), jnp.int32),
            jax.ShapeDtypeStruct((N,), jnp.float32),
            jax.ShapeDtypeStruct((N,), jnp.float32),
        ],
        mesh=plsc.VectorSubcoreMesh(core_axis_name="c", subcore_axis_name="s",
                                    num_cores=NC, num_subcores=NS),
        scratch_types=[
            pltpu.VMEM((E, PER_W), jnp.float32),
            pltpu.VMEM((PER_W,), jnp.int32),
            pltpu.VMEM((PER_W,), jnp.int32),
            pltpu.VMEM((PER_W,), jnp.float32),
            pltpu.VMEM((PER_W,), jnp.float32),
        ],
    )


@jax.jit
def kernel(routing_features, W1, b1, W2, b2, W3, b3, expert_bias, temperature):
    lt = _tc_logits(routing_features, W1, b1, W2, b2, W3, b3, expert_bias,
                    temperature)
    i1, i2, w1, w2 = _sc_top2()(lt)
    top_indices = jnp.stack([i1, i2], axis=-1)
    top_weights = jnp.stack([w1, w2], axis=-1)
    return (top_indices, top_weights)


# speedup vs baseline: 1.1411x; 1.1411x over previous
import functools

import jax
import jax.numpy as jnp
from jax import lax
from jax.experimental import pallas as pl
from jax.experimental.pallas import tpu as pltpu
from jax.experimental.pallas import tpu_sc as plsc

N = 32768
D = 128
E = 64
BT = 8192

NC = 2
NS = 16
L = 16
NW = NC * NS
PER_W = N // NW
GROUPS = PER_W // L


def _gelu_exact(x):
    return 0.5 * x * (1.0 + lax.erf(x * (2.0 ** -0.5)))


def _logits_block(x_ref, w1_ref, b1_ref, w2_ref, b2_ref, w3_ref, b3_ref,
                  eb_ref, temp_ref, out_ref):
    x = x_ref[...]
    h = jnp.dot(x, w1_ref[...]) + b1_ref[...]
    h = _gelu_exact(h)
    h = jnp.dot(h, w2_ref[...]) + b2_ref[...]
    h = _gelu_exact(h)
    lt = lax.dot_general(w3_ref[...], h, (((0,), (1,)), ((), ())))
    lt = lt + (b3_ref[...] + eb_ref[...])[:, None]
    inv_t = 1.0 / jnp.maximum(temp_ref[0], 0.1)
    out_ref[...] = lt * inv_t


def _tc_logits(routing_features, W1, b1, W2, b2, W3, b3, expert_bias,
               temperature):
    full = lambda i: (0, 0)
    return pl.pallas_call(
        _logits_block,
        grid=(N // BT,),
        in_specs=[
            pl.BlockSpec((BT, D), lambda i: (i, 0)),
            pl.BlockSpec((D, 2 * D), full),
            pl.BlockSpec((2 * D,), lambda i: (0,)),
            pl.BlockSpec((2 * D, D), full),
            pl.BlockSpec((D,), lambda i: (0,)),
            pl.BlockSpec((D, E), full),
            pl.BlockSpec((E,), lambda i: (0,)),
            pl.BlockSpec((E,), lambda i: (0,)),
            pl.BlockSpec((1,), lambda i: (0,)),
        ],
        out_specs=pl.BlockSpec((E, BT), lambda i: (0, i)),
        out_shape=jax.ShapeDtypeStruct((E, N), jnp.float32),
        compiler_params=pltpu.CompilerParams(
            dimension_semantics=("parallel",)),
    )(routing_features, W1, b1, W2, b2, W3, b3, expert_bias, temperature)


def _sc_top2_body(lt_hbm, i1_hbm, i2_hbm, w1_hbm, w2_hbm,
                  lv, i1v, i2v, w1v, w2v):
    wid = lax.axis_index("s") * NC + lax.axis_index("c")
    base = wid * PER_W
    pltpu.sync_copy(lt_hbm.at[:, pl.ds(base, PER_W)], lv)

    def top2_group(sl):
        m1 = lv[0, sl]
        i1 = jnp.zeros((L,), jnp.int32)
        m2 = jnp.full((L,), -jnp.inf, jnp.float32)
        i2 = jnp.zeros((L,), jnp.int32)
        for e in range(1, E):
            v = lv[e, sl]
            ei = jnp.full((L,), e, jnp.int32)
            gt1 = v > m1
            gt2 = v > m2
            m2 = jnp.maximum(m2, jnp.minimum(m1, v))
            i2 = jnp.where(gt1, i1, jnp.where(gt2, ei, i2))
            m1 = jnp.maximum(m1, v)
            i1 = jnp.where(gt1, ei, i1)
        s = jnp.exp(m2 - m1)
        d = 1.0 + s
        i1v[sl] = i1
        i2v[sl] = i2
        w1v[sl] = 1.0 / d
        w2v[sl] = s / d

    def group(g, carry):
        base2 = pl.multiple_of(g * (2 * L), 2 * L)
        top2_group(pl.ds(base2, L))
        top2_group(pl.ds(base2 + L, L))
        return carry

    lax.fori_loop(0, GROUPS // 2, group, 0)

    pltpu.sync_copy(i1v, i1_hbm.at[pl.ds(base, PER_W)])
    pltpu.sync_copy(i2v, i2_hbm.at[pl.ds(base, PER_W)])
    pltpu.sync_copy(w1v, w1_hbm.at[pl.ds(base, PER_W)])
    pltpu.sync_copy(w2v, w2_hbm.at[pl.ds(base, PER_W)])


@functools.cache
def _sc_top2():
    return pl.kernel(
        _sc_top2_body,
        out_type=[
            jax.ShapeDtypeStruct((N,), jnp.int32),
            jax.ShapeDtypeStruct((N,), jnp.int32),
            jax.ShapeDtypeStruct((N,), jnp.float32),
            jax.ShapeDtypeStruct((N,), jnp.float32),
        ],
        mesh=plsc.VectorSubcoreMesh(core_axis_name="c", subcore_axis_name="s",
                                    num_cores=NC, num_subcores=NS),
        scratch_types=[
            pltpu.VMEM((E, PER_W), jnp.float32),
            pltpu.VMEM((PER_W,), jnp.int32),
            pltpu.VMEM((PER_W,), jnp.int32),
            pltpu.VMEM((PER_W,), jnp.float32),
            pltpu.VMEM((PER_W,), jnp.float32),
        ],
    )


@jax.jit
def kernel(routing_features, W1, b1, W2, b2, W3, b3, expert_bias, temperature):
    lt = _tc_logits(routing_features, W1, b1, W2, b2, W3, b3, expert_bias,
                    temperature)
    i1, i2, w1, w2 = _sc_top2()(lt)
    top_indices = jnp.stack([i1, i2], axis=-1)
    top_weights = jnp.stack([w1, w2], axis=-1)
    return (top_indices, top_weights)
